# baseline (speedup 1.0000x reference)
import functools

import jax
import jax.numpy as jnp
from jax import lax
from jax.experimental import pallas as pl
from jax.experimental.pallas import tpu as pltpu
from jax.experimental.pallas import tpu_sc as plsc

_NC = 2
_NS = 16
_NW = _NC * _NS


def _sc_mesh():
    return plsc.VectorSubcoreMesh(
        core_axis_name="c", subcore_axis_name="s", num_cores=_NC,
        num_subcores=_NS)


def _worker_id():
    return lax.axis_index("s") * _NC + lax.axis_index("c")


def _sc_gather(weight, set_indices):
    B = set_indices.shape[0]
    D = weight.shape[1]
    CH = 1024
    NCH = pl.cdiv(B, CH)
    SLOTS = pl.cdiv(NCH, _NW)
    IDXV = 128

    @functools.partial(
        pl.kernel,
        out_type=jax.ShapeDtypeStruct((B, D), jnp.float32),
        mesh=_sc_mesh(),
        compiler_params=pltpu.CompilerParams(use_tc_tiling_on_sc=False),
        scratch_types=[
            pltpu.VMEM((CH,), jnp.int32),
            pltpu.VMEM((CH, D), jnp.float32),
            pltpu.SemaphoreType.DMA,
        ],
    )
    def gather_k(w_hbm, idx_hbm, out_hbm, idx_v, rows_v, gsem):
        wid = _worker_id()
        for slot in range(SLOTS):
            c = wid + slot * _NW

            @pl.when(c < NCH)
            def _():
                start = jnp.minimum(c * CH, B - CH)
                start = pl.multiple_of(start, 8)
                pltpu.sync_copy(idx_hbm.at[pl.ds(start, CH)], idx_v)
                cps = []
                for j in range(CH // IDXV):
                    cps.append(pltpu.async_copy(
                        w_hbm.at[idx_v.at[pl.ds(j * IDXV, IDXV)]],
                        rows_v.at[pl.ds(j * IDXV, IDXV)],
                        gsem))
                for cp in cps:
                    cp.wait()
                pltpu.sync_copy(rows_v, out_hbm.at[pl.ds(start, CH)])

    return gather_k(weight, set_indices)


def _tc_argmax(queries, keys):
    NQ, D = queries.shape
    B = keys.shape[0]
    BK = 2048
    G = pl.cdiv(B, BK)

    def body(q_ref, k_ref, o_ref, bval, bidx):
        i = pl.program_id(0)
        q = q_ref[...]
        k = k_ref[...]
        dn = (((1,), (1,)), ((), ()))
        raw = lax.dot_general(q, k, dn, preferred_element_type=jnp.float32,
                              precision=lax.Precision.HIGHEST)
        nsq = lax.dot_general(jnp.ones((1, D), jnp.float32), k * k, dn,
                              preferred_element_type=jnp.float32,
                              precision=lax.Precision.HIGHEST)
        rn = 1.0 / jnp.maximum(jnp.sqrt(nsq), 1e-12)
        gid = i * BK + lax.broadcasted_iota(jnp.int32, (NQ, BK), 1)
        s = jnp.where(gid < B, raw * rn, -jnp.inf)
        m = jnp.max(s, axis=1, keepdims=True)
        cidx = jnp.min(jnp.where(s == m, gid, jnp.int32(B)), axis=1,
                       keepdims=True)

        @pl.when(i == 0)
        def _():
            bval[...] = jnp.full((NQ, 1), -jnp.inf, jnp.float32)
            bidx[...] = jnp.zeros((NQ, 1), jnp.int32)

        upd = m > bval[...]
        bval[...] = jnp.where(upd, m, bval[...])
        bidx[...] = jnp.where(upd, cidx, bidx[...])

        @pl.when(i == G - 1)
        def _():
            o_ref[...] = bidx[...]

    return pl.pallas_call(
        body,
        grid=(G,),
        in_specs=[
            pl.BlockSpec((NQ, D), lambda i: (0, 0)),
            pl.BlockSpec((BK, D), lambda i: (i, 0)),
        ],
        out_specs=pl.BlockSpec((NQ, 1), lambda i: (0, 0)),
        out_shape=jax.ShapeDtypeStruct((NQ, 1), jnp.int32),
        scratch_shapes=[pltpu.VMEM((NQ, 1), jnp.float32),
                        pltpu.VMEM((NQ, 1), jnp.int32)],
    )(queries, keys)


def _sc_finalize(weight, set_indices, argidx):
    NQ = argidx.shape[0]
    D = weight.shape[1]

    @functools.partial(
        pl.kernel,
        out_type=(jax.ShapeDtypeStruct((NQ, D), jnp.float32),
                  jax.ShapeDtypeStruct((NQ,), jnp.int32)),
        mesh=_sc_mesh(),
        compiler_params=pltpu.CompilerParams(use_tc_tiling_on_sc=False),
        scratch_types=[
            pltpu.VMEM((NQ,), jnp.int32),
            pltpu.VMEM((NQ,), jnp.int32),
            pltpu.VMEM((NQ, D), jnp.float32),
            pltpu.SemaphoreType.DMA,
        ],
    )
    def fin(w_hbm, sidx_hbm, aidx_hbm, oemb_hbm, oidx_hbm, av, fv, ev, sem):
        @pl.when(_worker_id() == 0)
        def _():
            pltpu.sync_copy(aidx_hbm, av)
            pltpu.async_copy(sidx_hbm.at[av], fv, sem).wait()
            pltpu.async_copy(w_hbm.at[fv], ev, sem).wait()
            pltpu.sync_copy(fv, oidx_hbm)
            pltpu.sync_copy(ev, oemb_hbm)

    return fin(weight, set_indices, argidx)


def kernel(embedded_inputs, embedding_weight, set_indices, topk):
    bsz, seq_len, emb_dim = embedded_inputs.shape
    queries = embedded_inputs.reshape(-1, emb_dim)
    keys = _sc_gather(embedding_weight, set_indices)
    argidx = _tc_argmax(queries, keys).reshape(-1)
    emb, full = _sc_finalize(embedding_weight, set_indices, argidx)
    return emb.reshape(bsz, seq_len, emb_dim), full.reshape(bsz, seq_len)


# speedup vs baseline: 1.3371x; 1.3371x over previous
import functools

import jax
import jax.numpy as jnp
from jax import lax
from jax.experimental import pallas as pl
from jax.experimental.pallas import tpu as pltpu
from jax.experimental.pallas import tpu_sc as plsc

_NC = 2
_NS = 16
_NW = _NC * _NS


def _sc_mesh():
    return plsc.VectorSubcoreMesh(
        core_axis_name="c", subcore_axis_name="s", num_cores=_NC,
        num_subcores=_NS)


def _worker_id():
    return lax.axis_index("s") * _NC + lax.axis_index("c")


def _sc_gather(weight, set_indices):
    B = set_indices.shape[0]
    D = weight.shape[1]
    CH = 512
    K = 16
    NCH = pl.cdiv(B, CH)
    SLOTS = pl.cdiv(NCH, _NW)

    @functools.partial(
        pl.kernel,
        out_type=jax.ShapeDtypeStruct((B, D), jnp.float32),
        mesh=_sc_mesh(),
        compiler_params=pltpu.CompilerParams(use_tc_tiling_on_sc=True),
        scratch_types=[
            pltpu.VMEM((CH,), jnp.int32),
            pltpu.VMEM((CH, D), jnp.float32),
            pltpu.SemaphoreType.DMA,
        ],
    )
    def gather_k(w_hbm, idx_hbm, out_hbm, idx_v, rows_v, gsem):
        wid = _worker_id()
        for slot in range(SLOTS):
            c = wid + slot * _NW

            @pl.when(c < NCH)
            def _():
                start = jnp.minimum(c * CH, B - CH)
                start = pl.multiple_of(start, 8)
                pltpu.sync_copy(idx_hbm.at[pl.ds(start, CH)], idx_v)

                def burst(i, _):
                    vec = idx_v[pl.ds(i * K, K)]
                    cps = []
                    for j in range(K):
                        row = vec[j]
                        cps.append(pltpu.async_copy(
                            w_hbm.at[pl.ds(row, 1)],
                            rows_v.at[pl.ds(i * K + j, 1)],
                            gsem))
                    for cp in cps:
                        cp.wait()
                    return _

                lax.fori_loop(0, CH // K, burst, None)
                pltpu.sync_copy(rows_v, out_hbm.at[pl.ds(start, CH)])

    return gather_k(weight, set_indices)


def _tc_argmax(queries, keys):
    NQ, D = queries.shape
    B = keys.shape[0]
    BK = 2048
    G = pl.cdiv(B, BK)

    def body(q_ref, k_ref, o_ref, bval, bidx):
        i = pl.program_id(0)
        q = q_ref[...]
        k = k_ref[...]
        qn = q / jnp.maximum(
            jnp.sqrt(jnp.sum(q * q, axis=1, keepdims=True)), 1e-12)
        kn = k / jnp.maximum(
            jnp.sqrt(jnp.sum(k * k, axis=1, keepdims=True)), 1e-12)
        dn = (((1,), (1,)), ((), ()))
        s = lax.dot_general(qn.astype(jnp.bfloat16), kn.astype(jnp.bfloat16),
                            dn, preferred_element_type=jnp.float32)
        gid = i * BK + lax.broadcasted_iota(jnp.int32, (NQ, BK), 1)
        s = jnp.where(gid < B, s, -jnp.inf)
        m = jnp.max(s, axis=1, keepdims=True)
        cidx = jnp.min(jnp.where(s == m, gid, jnp.int32(B)), axis=1,
                       keepdims=True)

        @pl.when(i == 0)
        def _():
            bval[...] = jnp.full((NQ, 1), -jnp.inf, jnp.float32)
            bidx[...] = jnp.zeros((NQ, 1), jnp.int32)

        upd = m > bval[...]
        bval[...] = jnp.where(upd, m, bval[...])
        bidx[...] = jnp.where(upd, cidx, bidx[...])

        @pl.when(i == G - 1)
        def _():
            o_ref[...] = bidx[...]

    return pl.pallas_call(
        body,
        grid=(G,),
        in_specs=[
            pl.BlockSpec((NQ, D), lambda i: (0, 0)),
            pl.BlockSpec((BK, D), lambda i: (i, 0)),
        ],
        out_specs=pl.BlockSpec((NQ, 1), lambda i: (0, 0)),
        out_shape=jax.ShapeDtypeStruct((NQ, 1), jnp.int32),
        scratch_shapes=[pltpu.VMEM((NQ, 1), jnp.float32),
                        pltpu.VMEM((NQ, 1), jnp.int32)],
    )(queries, keys)


def kernel(embedded_inputs, embedding_weight, set_indices, topk):
    bsz, seq_len, emb_dim = embedded_inputs.shape
    queries = embedded_inputs.reshape(-1, emb_dim)
    keys = _sc_gather(embedding_weight, set_indices)
    argidx = _tc_argmax(queries, keys).reshape(-1)
    full = jnp.take(set_indices, argidx)
    emb = jnp.take(embedding_weight, full, axis=0)
    return emb.reshape(bsz, seq_len, emb_dim), full.reshape(bsz, seq_len)
